# baseline (device time: 120510 ns/iter reference)
import jax
import jax.numpy as jnp
from jax import lax
from jax.experimental import pallas as pl
from jax.experimental.pallas import tpu as pltpu

N_DEV = 32
N_ROW = 8
N_COL = 4
P = 2
NDIR = 2


def kernel(t, W):
    m, k = t.shape
    _, n = W.shape
    c1 = m // N_ROW
    c2 = c1 // N_COL
    hw = k // 2
    w = hw // P

    def col0(dir_, p):
        return dir_ * hw + p * w

    def body(t_hbm, w_hbm, out_ref, red1, red2,
             s1_send, s1_recv, z2_recv, t_ref, w_ref, io_sems,
             p1_ssem, p1_rsem, p2_ssem, p2_rsem,
             p3_ssem, p3_rsem, p4_ssem, p4_rsem):
        d = lax.axis_index("i")
        g = d // N_ROW
        r = d % N_ROW

        y = r // 2
        xe = r % 2
        x = jnp.where(y % 2 == 0, xe, 1 - xe)
        q = jnp.where(x == 1, y + 1, jnp.where(y == 0, 0, N_ROW - y))

        def r_of_q(qq):
            xq = jnp.where((qq >= 1) & (qq <= 4), 1, 0)
            yq = jnp.where(qq == 0, 0, jnp.where(qq <= 4, qq - 1, N_ROW - qq))
            return 2 * yq + jnp.where(yq % 2 == 0, xq, 1 - xq)

        row_right = g * N_ROW + r_of_q((q + 1) % N_ROW)
        row_left = g * N_ROW + r_of_q((q - 1) % N_ROW)
        row_nbr = (row_right, row_left)

        def z_peer(a):
            return ((g + a) % N_COL) * N_ROW + r

        rho1 = (q + 1) % N_ROW
        gam1 = (g + 1) % N_COL

        def ring_rdma(src, dst, ssem, rsem, target):
            return pltpu.make_async_remote_copy(
                src_ref=src, dst_ref=dst, send_sem=ssem, recv_sem=rsem,
                device_id=(target,), device_id_type=pl.DeviceIdType.MESH,
            )

        def p1_chunk(dir_, s):
            return (q - s) % N_ROW if dir_ == 0 else (q + s + 2) % N_ROW

        def t_piece(rho, dir_, p):
            return t_ref[pl.ds(rho * c1, c1), pl.ds(col0(dir_, p), w)]

        def p1_rdma(dir_, s, p):
            return ring_rdma(
                s1_send.at[dir_, s, p], s1_recv.at[dir_, s, p],
                p1_ssem.at[dir_, s, p], p1_rsem.at[dir_, s, p],
                row_nbr[dir_])

        def p2_send_rdma(dir_, a):
            gam = (g + a + 1) % N_COL
            return ring_rdma(
                red1.at[pl.ds(gam * c2, c2), pl.ds(dir_ * hw, hw)],
                z2_recv.at[dir_, N_COL - a],
                p2_ssem.at[dir_, a], p2_rsem.at[dir_, N_COL - a],
                z_peer(a))

        def p2_recv_rdma(dir_, jj):
            return ring_rdma(
                z2_recv.at[dir_, jj], z2_recv.at[dir_, jj],
                p2_ssem.at[dir_, jj], p2_rsem.at[dir_, jj],
                z_peer(jj))

        own_rows = pl.ds(rho1 * c1 + gam1 * c2, c2)

        def p3_send_rdma(dir_, a):
            sl = (own_rows, pl.ds(dir_ * hw, hw))
            return ring_rdma(
                out_ref.at[sl], out_ref.at[sl],
                p3_ssem.at[dir_, a], p3_rsem.at[dir_, N_COL - a],
                z_peer(a))

        def p3_recv_rdma(dir_, jj):
            gam = (g + jj + 1) % N_COL
            sl = (pl.ds(rho1 * c1 + gam * c2, c2), pl.ds(dir_ * hw, hw))
            return ring_rdma(
                out_ref.at[sl], out_ref.at[sl],
                p3_ssem.at[dir_, jj], p3_rsem.at[dir_, jj],
                z_peer(jj))

        def p4_rdma(dir_, h, p):
            rho = ((q + 1 - h) if dir_ == 0 else (q + h + 1)) % N_ROW
            sl = (pl.ds(rho * c1, c1), pl.ds(col0(dir_, p), w))
            return ring_rdma(
                out_ref.at[sl], out_ref.at[sl],
                p4_ssem.at[dir_, h, p], p4_rsem.at[dir_, h, p],
                row_nbr[dir_])

        stage_cps = []
        for dir_ in range(NDIR):
            for p in range(P):
                cp = pltpu.make_async_copy(
                    t_hbm.at[pl.ds(p1_chunk(dir_, 0) * c1, c1),
                             pl.ds(col0(dir_, p), w)],
                    s1_send.at[dir_, 0, p],
                    io_sems.at[dir_ * P + p],
                )
                cp.start()
                stage_cps.append(cp)
        t_cp = pltpu.make_async_copy(t_hbm, t_ref, io_sems.at[4])
        t_cp.start()
        w_cp = pltpu.make_async_copy(w_hbm, w_ref, io_sems.at[5])
        w_cp.start()
        for cp in stage_cps:
            cp.wait()

        barrier_sem = pltpu.get_barrier_semaphore()
        for nbr in (row_left, row_right, z_peer(1), z_peer(2), z_peer(3)):
            pl.semaphore_signal(
                barrier_sem, inc=1,
                device_id=(nbr,), device_id_type=pl.DeviceIdType.MESH,
            )
        pl.semaphore_wait(barrier_sem, 5)

        for dir_ in range(NDIR):
            for p in range(P):
                p1_rdma(dir_, 0, p).start()
        t_cp.wait()
        for s in range(N_ROW - 1):
            for p in range(P):
                for dir_ in range(NDIR):
                    rdma = p1_rdma(dir_, s, p)
                    rdma.wait_recv()
                    acc = s1_recv[dir_, s, p] + t_piece(
                        p1_chunk(dir_, s + 1), dir_, p)
                    if s < N_ROW - 2:
                        s1_send[dir_, s + 1, p] = acc
                        p1_rdma(dir_, s + 1, p).start()
                    else:
                        red1[:, pl.ds(col0(dir_, p), w)] = acc

        for a in range(1, N_COL):
            for dir_ in range(NDIR):
                p2_send_rdma(dir_, a).start()
        for dir_ in range(NDIR):
            for s in range(N_ROW - 1):
                for p in range(P):
                    p1_rdma(dir_, s, p).wait_send()
        for dir_ in range(NDIR):
            for jj in range(1, N_COL):
                p2_recv_rdma(dir_, jj).wait_recv()
            red2[:, pl.ds(dir_ * hw, hw)] = (
                (red1[pl.ds(gam1 * c2, c2), pl.ds(dir_ * hw, hw)]
                 + z2_recv[dir_, 1])
                + (z2_recv[dir_, 2] + z2_recv[dir_, 3])
            )

        w_cp.wait()
        for dir_ in range(NDIR):
            out_ref[own_rows, pl.ds(dir_ * hw, hw)] = jnp.dot(
                red2[:, :], w_ref[:, pl.ds(dir_ * hw, hw)],
                preferred_element_type=jnp.float32,
            )
            for a in range(1, N_COL):
                p3_send_rdma(dir_, a).start()

        for a in range(1, N_COL):
            for dir_ in range(NDIR):
                p2_send_rdma(dir_, a).wait_send()

        for dir_ in range(NDIR):
            for jj in range(1, N_COL):
                p3_recv_rdma(dir_, jj).wait_recv()
            for p in range(P):
                p4_rdma(dir_, 0, p).start()

        for a in range(1, N_COL):
            for dir_ in range(NDIR):
                p3_send_rdma(dir_, a).wait_send()

        for h in range(N_ROW - 1):
            for p in range(P):
                for dir_ in range(NDIR):
                    rdma = p4_rdma(dir_, h, p)
                    rdma.wait_recv()
                    if h < N_ROW - 2:
                        p4_rdma(dir_, h + 1, p).start()

        for dir_ in range(NDIR):
            for h in range(N_ROW - 1):
                for p in range(P):
                    p4_rdma(dir_, h, p).wait_send()

    return pl.pallas_call(
        body,
        out_shape=jax.ShapeDtypeStruct((m, n), jnp.float32),
        in_specs=[
            pl.BlockSpec(memory_space=pl.ANY),
            pl.BlockSpec(memory_space=pl.ANY),
        ],
        out_specs=pl.BlockSpec(memory_space=pltpu.VMEM),
        scratch_shapes=[
            pltpu.VMEM((c1, k), jnp.float32),
            pltpu.VMEM((c2, k), jnp.float32),
            pltpu.VMEM((NDIR, N_ROW - 1, P, c1, w), jnp.float32),
            pltpu.VMEM((NDIR, N_ROW - 1, P, c1, w), jnp.float32),
            pltpu.VMEM((NDIR, N_COL, c2, hw), jnp.float32),
            pltpu.VMEM((m, k), jnp.float32),
            pltpu.VMEM((k, n), jnp.float32),
            pltpu.SemaphoreType.DMA((6,)),
            pltpu.SemaphoreType.DMA((NDIR, N_ROW - 1, P)),
            pltpu.SemaphoreType.DMA((NDIR, N_ROW - 1, P)),
            pltpu.SemaphoreType.DMA((NDIR, N_COL)),
            pltpu.SemaphoreType.DMA((NDIR, N_COL)),
            pltpu.SemaphoreType.DMA((NDIR, N_COL)),
            pltpu.SemaphoreType.DMA((NDIR, N_COL)),
            pltpu.SemaphoreType.DMA((NDIR, N_ROW - 1, P)),
            pltpu.SemaphoreType.DMA((NDIR, N_ROW - 1, P)),
        ],
        compiler_params=pltpu.CompilerParams(collective_id=0),
    )(t, W)


# device time: 117735 ns/iter; 1.0236x vs baseline; 1.0236x over previous
import jax
import jax.numpy as jnp
from jax import lax
from jax.experimental import pallas as pl
from jax.experimental.pallas import tpu as pltpu

N_DEV = 32
N_ROW = 8
N_COL = 4
P = 2
NDIR = 2


def kernel(t, W):
    m, k = t.shape
    _, n = W.shape
    c1 = m // N_ROW
    c2 = c1 // N_COL
    hw = k // 2
    w = hw // P

    def col0(dir_, p):
        return dir_ * hw + p * w

    def body(t_hbm, w_hbm, out_ref, red1, red2,
             s1_send, s1_recv, z2_recv, t_ref, w_ref, io_sems,
             p1_ssem, p1_rsem, p2_ssem, p2_rsem,
             p3_ssem, p3_rsem, p4_ssem, p4_rsem):
        d = lax.axis_index("i")
        g = d // N_ROW
        r = d % N_ROW

        y = r // 2
        xe = r % 2
        x = jnp.where(y % 2 == 0, xe, 1 - xe)
        q = jnp.where(x == 1, y + 1, jnp.where(y == 0, 0, N_ROW - y))

        def r_of_q(qq):
            xq = jnp.where((qq >= 1) & (qq <= 4), 1, 0)
            yq = jnp.where(qq == 0, 0, jnp.where(qq <= 4, qq - 1, N_ROW - qq))
            return 2 * yq + jnp.where(yq % 2 == 0, xq, 1 - xq)

        row_right = g * N_ROW + r_of_q((q + 1) % N_ROW)
        row_left = g * N_ROW + r_of_q((q - 1) % N_ROW)
        row_nbr = (row_right, row_left)

        def z_peer(a):
            return ((g + a) % N_COL) * N_ROW + r

        rho1 = (q + 1) % N_ROW
        gam1 = (g + 1) % N_COL

        def ring_rdma(src, dst, ssem, rsem, target):
            return pltpu.make_async_remote_copy(
                src_ref=src, dst_ref=dst, send_sem=ssem, recv_sem=rsem,
                device_id=(target,), device_id_type=pl.DeviceIdType.MESH,
            )

        def p1_chunk(dir_, s):
            return (q - s) % N_ROW if dir_ == 0 else (q + s + 2) % N_ROW

        def t_piece(rho, dir_, p):
            return t_ref[pl.ds(rho * c1, c1), pl.ds(col0(dir_, p), w)]

        def p1_rdma(dir_, s, p):
            return ring_rdma(
                s1_send.at[dir_, s, p], s1_recv.at[dir_, s, p],
                p1_ssem.at[dir_, s, p], p1_rsem.at[dir_, s, p],
                row_nbr[dir_])

        def p2_send_rdma(dir_, a):
            gam = (g + a + 1) % N_COL
            return ring_rdma(
                red1.at[pl.ds(gam * c2, c2), pl.ds(dir_ * hw, hw)],
                z2_recv.at[dir_, N_COL - a],
                p2_ssem.at[dir_, a], p2_rsem.at[dir_, N_COL - a],
                z_peer(a))

        def p2_recv_rdma(dir_, jj):
            return ring_rdma(
                z2_recv.at[dir_, jj], z2_recv.at[dir_, jj],
                p2_ssem.at[dir_, jj], p2_rsem.at[dir_, jj],
                z_peer(jj))

        own_rows = pl.ds(rho1 * c1 + gam1 * c2, c2)

        def p3_send_rdma(dir_, a):
            sl = (own_rows, pl.ds(dir_ * hw, hw))
            return ring_rdma(
                out_ref.at[sl], out_ref.at[sl],
                p3_ssem.at[dir_, a], p3_rsem.at[dir_, N_COL - a],
                z_peer(a))

        def p3_recv_rdma(dir_, jj):
            gam = (g + jj + 1) % N_COL
            sl = (pl.ds(rho1 * c1 + gam * c2, c2), pl.ds(dir_ * hw, hw))
            return ring_rdma(
                out_ref.at[sl], out_ref.at[sl],
                p3_ssem.at[dir_, jj], p3_rsem.at[dir_, jj],
                z_peer(jj))

        def p4_gam(slot):
            return gam1 if slot == 0 else (g + slot + 1) % N_COL

        def p4_rdma(dir_, h, slot):
            rho = ((q + 1 - h) if dir_ == 0 else (q + h + 1)) % N_ROW
            sl = (pl.ds(rho * c1 + p4_gam(slot) * c2, c2),
                  pl.ds(dir_ * hw, hw))
            return ring_rdma(
                out_ref.at[sl], out_ref.at[sl],
                p4_ssem.at[dir_, h, slot], p4_rsem.at[dir_, h, slot],
                row_nbr[dir_])

        stage_cps = []
        for dir_ in range(NDIR):
            for p in range(P):
                cp = pltpu.make_async_copy(
                    t_hbm.at[pl.ds(p1_chunk(dir_, 0) * c1, c1),
                             pl.ds(col0(dir_, p), w)],
                    s1_send.at[dir_, 0, p],
                    io_sems.at[dir_ * P + p],
                )
                cp.start()
                stage_cps.append(cp)
        t_cp = pltpu.make_async_copy(t_hbm, t_ref, io_sems.at[4])
        t_cp.start()
        w_cp = pltpu.make_async_copy(w_hbm, w_ref, io_sems.at[5])
        w_cp.start()
        for cp in stage_cps:
            cp.wait()

        barrier_sem = pltpu.get_barrier_semaphore()
        for nbr in (row_left, row_right, z_peer(1), z_peer(2), z_peer(3)):
            pl.semaphore_signal(
                barrier_sem, inc=1,
                device_id=(nbr,), device_id_type=pl.DeviceIdType.MESH,
            )
        pl.semaphore_wait(barrier_sem, 5)

        for dir_ in range(NDIR):
            for p in range(P):
                p1_rdma(dir_, 0, p).start()
        t_cp.wait()
        for s in range(N_ROW - 1):
            for p in range(P):
                for dir_ in range(NDIR):
                    rdma = p1_rdma(dir_, s, p)
                    rdma.wait_recv()
                    acc = s1_recv[dir_, s, p] + t_piece(
                        p1_chunk(dir_, s + 1), dir_, p)
                    if s < N_ROW - 2:
                        s1_send[dir_, s + 1, p] = acc
                        p1_rdma(dir_, s + 1, p).start()
                    else:
                        red1[:, pl.ds(col0(dir_, p), w)] = acc

        for a in range(1, N_COL):
            for dir_ in range(NDIR):
                p2_send_rdma(dir_, a).start()
        for dir_ in range(NDIR):
            for s in range(N_ROW - 1):
                for p in range(P):
                    p1_rdma(dir_, s, p).wait_send()
        for dir_ in range(NDIR):
            for jj in range(1, N_COL):
                p2_recv_rdma(dir_, jj).wait_recv()
            red2[:, pl.ds(dir_ * hw, hw)] = (
                (red1[pl.ds(gam1 * c2, c2), pl.ds(dir_ * hw, hw)]
                 + z2_recv[dir_, 1])
                + (z2_recv[dir_, 2] + z2_recv[dir_, 3])
            )

        w_cp.wait()
        for dir_ in range(NDIR):
            out_ref[own_rows, pl.ds(dir_ * hw, hw)] = jnp.dot(
                red2[:, :], w_ref[:, pl.ds(dir_ * hw, hw)],
                preferred_element_type=jnp.float32,
            )
            for a in range(1, N_COL):
                p3_send_rdma(dir_, a).start()
            p4_rdma(dir_, 0, 0).start()

        for a in range(1, N_COL):
            for dir_ in range(NDIR):
                p2_send_rdma(dir_, a).wait_send()

        for dir_ in range(NDIR):
            for jj in range(1, N_COL):
                p3_recv_rdma(dir_, jj).wait_recv()
                p4_rdma(dir_, 0, jj).start()

        for a in range(1, N_COL):
            for dir_ in range(NDIR):
                p3_send_rdma(dir_, a).wait_send()

        for h in range(N_ROW - 1):
            for slot in range(N_COL):
                for dir_ in range(NDIR):
                    rdma = p4_rdma(dir_, h, slot)
                    rdma.wait_recv()
                    if h < N_ROW - 2:
                        p4_rdma(dir_, h + 1, slot).start()

        for dir_ in range(NDIR):
            for h in range(N_ROW - 1):
                for slot in range(N_COL):
                    p4_rdma(dir_, h, slot).wait_send()

    return pl.pallas_call(
        body,
        out_shape=jax.ShapeDtypeStruct((m, n), jnp.float32),
        in_specs=[
            pl.BlockSpec(memory_space=pl.ANY),
            pl.BlockSpec(memory_space=pl.ANY),
        ],
        out_specs=pl.BlockSpec(memory_space=pltpu.VMEM),
        scratch_shapes=[
            pltpu.VMEM((c1, k), jnp.float32),
            pltpu.VMEM((c2, k), jnp.float32),
            pltpu.VMEM((NDIR, N_ROW - 1, P, c1, w), jnp.float32),
            pltpu.VMEM((NDIR, N_ROW - 1, P, c1, w), jnp.float32),
            pltpu.VMEM((NDIR, N_COL, c2, hw), jnp.float32),
            pltpu.VMEM((m, k), jnp.float32),
            pltpu.VMEM((k, n), jnp.float32),
            pltpu.SemaphoreType.DMA((6,)),
            pltpu.SemaphoreType.DMA((NDIR, N_ROW - 1, P)),
            pltpu.SemaphoreType.DMA((NDIR, N_ROW - 1, P)),
            pltpu.SemaphoreType.DMA((NDIR, N_COL)),
            pltpu.SemaphoreType.DMA((NDIR, N_COL)),
            pltpu.SemaphoreType.DMA((NDIR, N_COL)),
            pltpu.SemaphoreType.DMA((NDIR, N_COL)),
            pltpu.SemaphoreType.DMA((NDIR, N_ROW - 1, N_COL)),
            pltpu.SemaphoreType.DMA((NDIR, N_ROW - 1, N_COL)),
        ],
        compiler_params=pltpu.CompilerParams(collective_id=0),
    )(t, W)
